# baseline (speedup 1.0000x reference)
import functools
import math

import numpy as np
import jax
import jax.numpy as jnp
from jax import lax
from jax.experimental import pallas as pl
from jax.experimental.pallas import tpu as pltpu

_ISIZE = 224
_PSIZE = 30
_rng = np.random.RandomState(0)
_X = int(_rng.randint(0, _ISIZE - _PSIZE))
_Y = int(_rng.randint(0, _ISIZE - _PSIZE))

_ROWS = 3 * _ISIZE * _ISIZE // 128
_CB = 4
_R = 3


def _canvas_kernel(p_ref, c_ref):
    c_ref[...] = jnp.zeros_like(c_ref)
    c_ref[:, :, _X:_X + _PSIZE, _Y:_Y + _PSIZE] = p_ref[...]


def _make_add_body(num_cores, n_chunks):
    per_core = n_chunks // num_cores

    def body(x_hbm, c_hbm, o_hbm, cvs, in_bufs, out_bufs,
             in_sems, out_sems, c_sem):
        core = lax.axis_index("core")
        base = core * (per_core * _CB)

        pltpu.make_async_copy(c_hbm, cvs, c_sem).start()
        pltpu.make_async_copy(c_hbm, cvs, c_sem).wait()

        def in_copy(c, b):
            return pltpu.make_async_copy(
                x_hbm.at[pl.ds(base + c * _CB, _CB)],
                in_bufs.at[b], in_sems.at[b])

        def out_copy(c, b):
            return pltpu.make_async_copy(
                out_bufs.at[b],
                o_hbm.at[pl.ds(base + c * _CB, _CB)], out_sems.at[b])

        for c in range(min(_R, per_core)):
            in_copy(c, c % _R).start()
        for c in range(per_core):
            b = c % _R
            in_copy(c, b).wait()
            if c >= _R:
                out_copy(c - _R, b).wait()
            out_bufs[b] = in_bufs[b] + cvs[...]
            out_copy(c, b).start()
            if c + _R < per_core:
                in_copy(c + _R, b).start()
        for c in range(max(0, per_core - _R), per_core):
            out_copy(c, c % _R).wait()

    return body


def kernel(x, patch):
    B = x.shape[0]
    canvas = pl.pallas_call(
        _canvas_kernel,
        out_shape=jax.ShapeDtypeStruct((1, 3, _ISIZE, _ISIZE), x.dtype),
    )(patch)
    x2 = x.reshape(B, _ROWS, 128)
    c2 = canvas.reshape(1, _ROWS, 128)

    mesh = pltpu.create_tensorcore_mesh("core")
    num_cores = math.prod(mesh.shape.values())
    n_chunks = B // _CB

    out = pl.kernel(
        _make_add_body(num_cores, n_chunks),
        out_type=jax.ShapeDtypeStruct((B, _ROWS, 128), x.dtype),
        mesh=mesh,
        scratch_types=[
            pltpu.VMEM((1, _ROWS, 128), x.dtype),
            pltpu.VMEM((_R, _CB, _ROWS, 128), x.dtype),
            pltpu.VMEM((_R, _CB, _ROWS, 128), x.dtype),
            pltpu.SemaphoreType.DMA((_R,)),
            pltpu.SemaphoreType.DMA((_R,)),
            pltpu.SemaphoreType.DMA,
        ],
        compiler_params=pltpu.CompilerParams(
            vmem_limit_bytes=100 * 1024 * 1024),
    )(x2, c2)
    return out.reshape(x.shape)
